# Initial kernel scaffold; baseline (speedup 1.0000x reference)
import jax
import jax.numpy as jnp
from jax.experimental import pallas as pl


def kernel(stale_features, staleness, pose_delta, class_indices, valid_mask, prototypes, W1, b1, W2, b2):
    raise NotImplementedError("write your pallas kernel here")


# speedup vs baseline: 1.0007x; 1.0007x over previous
import functools

import jax
import jax.numpy as jnp
from jax import lax
from jax.experimental import pallas as pl
from jax.experimental.pallas import tpu as pltpu
from jax.experimental.pallas import tpu_sc as plsc

N = 16384
D = 256
NW = 32
RW = N // NW
C = 64
NCHUNK = RW // C
NG = C // 16
LN2 = 0.6931471805599453
SQRT2 = 1.4142135623730951


def _ln(x):
    bits = lax.bitcast_convert_type(x, jnp.int32)
    e = ((bits >> 23) & 0xFF) - 127
    m = lax.bitcast_convert_type((bits & 0x7FFFFF) | 0x3F800000, jnp.float32)
    big = m > SQRT2
    m = jnp.where(big, m * 0.5, m)
    e = jnp.where(big, e + 1, e)
    s = (m - 1.0) / (m + 1.0)
    s2 = s * s
    p = (2.0 * s) * (1.0 + s2 * ((1.0 / 3.0) + s2 * (0.2 + s2 * (1.0 / 7.0))))
    return e.astype(jnp.float32) * LN2 + p


def _bcast(vec, i):
    return vec.at[jnp.full((16,), i, jnp.int32)].get(mode="promise_in_bounds")


def _body(stale_hbm, stal_hbm, pose_hbm, ci_hbm, vm_hbm, proto_hbm,
          params_hbm, out_hbm,
          idx_v, stal_v, pose_v, mask_v, a_v, b_v, proto_v, rows_v,
          params_v, sem):
    wid = lax.axis_index("s") * 2 + lax.axis_index("c")
    pltpu.sync_copy(params_hbm, params_v)

    def chunk_body(k, carry):
        base = wid * RW + k * C
        pltpu.sync_copy(ci_hbm.at[pl.ds(base, C)], idx_v)
        gat = pltpu.async_copy(proto_hbm.at[idx_v], proto_v, sem)
        pltpu.sync_copy(stal_hbm.at[pl.ds(base, C)], stal_v)
        pltpu.sync_copy(pose_hbm.at[pl.ds(base, C)], pose_v)
        pltpu.sync_copy(vm_hbm.at[pl.ds(base, C)], mask_v)
        pltpu.sync_copy(stale_hbm.at[pl.ds(base, C)], rows_v)

        w10v = params_v[pl.ds(0, 16)]
        w11v = params_v[pl.ds(16, 16)]
        b1v = params_v[pl.ds(32, 16)]
        w2v = params_v[pl.ds(48, 16)]
        b2 = _bcast(params_v[pl.ds(64, 16)], 0)
        x0s, x1s, accs = [], [], []
        for g in range(NG):
            sl = pl.ds(g * 16, 16)
            x0s.append(_ln(stal_v[sl] + 1.0))
            x1s.append(pose_v[sl])
            accs.append(b2)
        for j in range(16):
            wa = _bcast(w10v, j)
            wb = _bcast(w11v, j)
            wc = _bcast(b1v, j)
            wo = _bcast(w2v, j)
            for g in range(NG):
                h = jnp.maximum(x0s[g] * wa + x1s[g] * wb + wc, 0.0)
                accs[g] = accs[g] + h * wo
        for g in range(NG):
            sl = pl.ds(g * 16, 16)
            conf = 1.0 / (1.0 + jnp.exp(-accs[g]))
            m = mask_v[sl]
            cm = conf * m
            a_v[sl] = cm
            b_v[sl] = m - cm

        gat.wait()

        for g in range(NG):
            a16 = a_v[pl.ds(g * 16, 16)]
            b16 = b_v[pl.ds(g * 16, 16)]

            def row_body(i, rc, g=g, a16=a16, b16=b16):
                r = g * 16 + i
                a = _bcast(a16, i)
                b = _bcast(b16, i)
                for cg in range(D // 16):
                    cs = pl.ds(cg * 16, 16)
                    rows_v[r, cs] = a * rows_v[r, cs] + b * proto_v[r, cs]
                return rc
            lax.fori_loop(0, 16, row_body, 0)

        pltpu.sync_copy(rows_v, out_hbm.at[pl.ds(base, C)])
        return carry

    lax.fori_loop(0, NCHUNK, chunk_body, 0)


@jax.jit
def _run(stale_features, staleness_f, pose_delta, class_indices, valid_f,
         prototypes, params):
    mesh = plsc.VectorSubcoreMesh(core_axis_name="c", subcore_axis_name="s")
    kern = functools.partial(
        pl.kernel,
        out_type=jax.ShapeDtypeStruct((N, D), jnp.float32),
        mesh=mesh,
        scratch_types=[
            pltpu.VMEM((C,), jnp.int32),
            pltpu.VMEM((C,), jnp.float32),
            pltpu.VMEM((C,), jnp.float32),
            pltpu.VMEM((C,), jnp.float32),
            pltpu.VMEM((C,), jnp.float32),
            pltpu.VMEM((C,), jnp.float32),
            pltpu.VMEM((C, D), jnp.float32),
            pltpu.VMEM((C, D), jnp.float32),
            pltpu.VMEM((80,), jnp.float32),
            pltpu.SemaphoreType.DMA,
        ],
    )(_body)
    return kern(stale_features, staleness_f, pose_delta, class_indices,
                valid_f, prototypes, params)


def kernel(stale_features, staleness, pose_delta, class_indices, valid_mask,
           prototypes, W1, b1, W2, b2):
    params = jnp.concatenate([
        W1.reshape(-1), b1.reshape(-1), W2.reshape(-1), b2.reshape(-1),
        jnp.zeros((15,), jnp.float32),
    ])
    return _run(stale_features, staleness.astype(jnp.float32), pose_delta,
                class_indices, valid_mask.astype(jnp.float32), prototypes,
                params)


# speedup vs baseline: 1.2507x; 1.2498x over previous
import functools

import jax
import jax.numpy as jnp
from jax import lax
from jax.experimental import pallas as pl
from jax.experimental.pallas import tpu as pltpu
from jax.experimental.pallas import tpu_sc as plsc

N = 16384
D = 256
NW = 32
RW = N // NW
C = 64
NCHUNK = RW // C
NG = C // 16
LN2 = 0.6931471805599453
SQRT2 = 1.4142135623730951


def _ln(x):
    bits = lax.bitcast_convert_type(x, jnp.int32)
    e = ((bits >> 23) & 0xFF) - 127
    m = lax.bitcast_convert_type((bits & 0x7FFFFF) | 0x3F800000, jnp.float32)
    big = m > SQRT2
    m = jnp.where(big, m * 0.5, m)
    e = jnp.where(big, e + 1, e)
    s = (m - 1.0) / (m + 1.0)
    s2 = s * s
    p = (2.0 * s) * (1.0 + s2 * ((1.0 / 3.0) + s2 * (0.2 + s2 * (1.0 / 7.0))))
    return e.astype(jnp.float32) * LN2 + p


def _bcast(vec, i):
    return vec.at[jnp.full((16,), i, jnp.int32)].get(mode="promise_in_bounds")


def _body(stale_hbm, stal_hbm, pose_hbm, ci_hbm, vm_hbm, proto_hbm,
          params_hbm, out_hbm,
          idx_v, stal_v, pose_v, mask_v,
          proto0, proto1, rows0, rows1, out0, out1,
          params_v,
          sg0, sg1, si0, si1, so0, so1):
    wid = lax.axis_index("s") * 2 + lax.axis_index("c")
    base = wid * RW
    protos = (proto0, proto1)
    rows = (rows0, rows1)
    outs = (out0, out1)
    sgs = (sg0, sg1)
    sis = (si0, si1)
    sos = (so0, so1)

    pltpu.sync_copy(params_hbm, params_v)
    pltpu.sync_copy(ci_hbm.at[pl.ds(base, RW)], idx_v)
    pltpu.sync_copy(stal_hbm.at[pl.ds(base, RW)], stal_v)
    pltpu.sync_copy(pose_hbm.at[pl.ds(base, RW)], pose_v)
    pltpu.sync_copy(vm_hbm.at[pl.ds(base, RW)], mask_v)

    w10v = params_v[pl.ds(0, 16)]
    w11v = params_v[pl.ds(16, 16)]
    b1v = params_v[pl.ds(32, 16)]
    w2v = params_v[pl.ds(48, 16)]
    b2 = _bcast(params_v[pl.ds(64, 16)], 0)

    def issue_in(k):
        s = k & 1
        g = pltpu.async_copy(proto_hbm.at[idx_v.at[pl.ds(k * C, C)]],
                             protos[s], sgs[s])
        r = pltpu.async_copy(stale_hbm.at[pl.ds(base + k * C, C)],
                             rows[s], sis[s])
        return g, r

    inflight = {0: issue_in(0)}
    out_dma = {}
    for k in range(NCHUNK):
        s = k & 1
        if k + 1 < NCHUNK:
            inflight[k + 1] = issue_in(k + 1)

        x0s, x1s, accs = [], [], []
        for g in range(NG):
            sl = pl.ds(k * C + g * 16, 16)
            x0s.append(_ln(stal_v[sl] + 1.0))
            x1s.append(pose_v[sl])
            accs.append(b2)

        def mlp_j(j, accs):
            wa = _bcast(w10v, j)
            wb = _bcast(w11v, j)
            wc = _bcast(b1v, j)
            wo = _bcast(w2v, j)
            return tuple(
                acc + wo * jnp.maximum(x0s[g] * wa + x1s[g] * wb + wc, 0.0)
                for g, acc in enumerate(accs))
        accs = lax.fori_loop(0, 16, mlp_j, tuple(accs))

        gin, rin = inflight.pop(k)
        gin.wait()
        rin.wait()
        if k >= 2:
            out_dma.pop(k - 2).wait()

        pv, rv, ov = protos[s], rows[s], outs[s]
        for g in range(NG):
            m = mask_v[pl.ds(k * C + g * 16, 16)]
            conf = 1.0 / (1.0 + jnp.exp(-accs[g]))
            cm = conf * m
            a16 = cm
            b16 = m - cm

            def row_body(i, rc, g=g, a16=a16, b16=b16):
                r = g * 16 + i
                a = _bcast(a16, i)
                b = _bcast(b16, i)
                for cg in range(D // 16):
                    cs = pl.ds(cg * 16, 16)
                    ov[r, cs] = a * rv[r, cs] + b * pv[r, cs]
                return rc
            lax.fori_loop(0, 16, row_body, 0)

        out_dma[k] = pltpu.async_copy(ov, out_hbm.at[pl.ds(base + k * C, C)],
                                      sos[s])
    for k in (NCHUNK - 2, NCHUNK - 1):
        out_dma.pop(k).wait()


@jax.jit
def _run(stale_features, staleness_f, pose_delta, class_indices, valid_f,
         prototypes, params):
    mesh = plsc.VectorSubcoreMesh(core_axis_name="c", subcore_axis_name="s")
    kern = functools.partial(
        pl.kernel,
        out_type=jax.ShapeDtypeStruct((N, D), jnp.float32),
        mesh=mesh,
        scratch_types=[
            pltpu.VMEM((RW,), jnp.int32),
            pltpu.VMEM((RW,), jnp.float32),
            pltpu.VMEM((RW,), jnp.float32),
            pltpu.VMEM((RW,), jnp.float32),
            pltpu.VMEM((C, D), jnp.float32),
            pltpu.VMEM((C, D), jnp.float32),
            pltpu.VMEM((C, D), jnp.float32),
            pltpu.VMEM((C, D), jnp.float32),
            pltpu.VMEM((C, D), jnp.float32),
            pltpu.VMEM((C, D), jnp.float32),
            pltpu.VMEM((80,), jnp.float32),
            pltpu.SemaphoreType.DMA,
            pltpu.SemaphoreType.DMA,
            pltpu.SemaphoreType.DMA,
            pltpu.SemaphoreType.DMA,
            pltpu.SemaphoreType.DMA,
            pltpu.SemaphoreType.DMA,
        ],
    )(_body)
    return kern(stale_features, staleness_f, pose_delta, class_indices,
                valid_f, prototypes, params)


def kernel(stale_features, staleness, pose_delta, class_indices, valid_mask,
           prototypes, W1, b1, W2, b2):
    params = jnp.concatenate([
        W1.reshape(-1), b1.reshape(-1), W2.reshape(-1), b2.reshape(-1),
        jnp.zeros((15,), jnp.float32),
    ])
    return _run(stale_features, staleness.astype(jnp.float32), pose_delta,
                class_indices, valid_mask.astype(jnp.float32), prototypes,
                params)
